# baseline (speedup 1.0000x reference)
import functools

import jax
import jax.numpy as jnp
import numpy as np
from jax import lax
from jax.experimental import pallas as pl
from jax.experimental.pallas import tpu as pltpu
from jax.experimental.pallas import tpu_sc as plsc

N = 10000
N_PAD = 10240
E = 320000
F = 128
FH = F // 2

NC = 2
NS = 16
NW = NC * NS

C = 80
CHUNKS_PER_TILE = E // (NS * C)
XROWS = N // NS
XCONV = 125

MM_BLOCK = 2000
CB_BLOCK = 2000

def _matmul_body(x_ref, w_ref, o_ref):
    o_ref[...] = jnp.dot(x_ref[...], w_ref[...],
                         preferred_element_type=jnp.float32)


def _matmul(X, W):
    return pl.pallas_call(
        _matmul_body,
        grid=(N // MM_BLOCK,),
        in_specs=[
            pl.BlockSpec((MM_BLOCK, F), lambda i: (i, 0)),
            pl.BlockSpec((F, F), lambda i: (0, 0)),
        ],
        out_specs=pl.BlockSpec((MM_BLOCK, F), lambda i: (i, 0)),
        out_shape=jax.ShapeDtypeStruct((N, F), jnp.float32),
    )(X, W)


def _sc_body(xw_hbm, packed_hbm, vals_hbm, b_hbm, out_hbm,
             packed_v, vals_v, sr0, sr1, sr2, sr3, dr0, dr1, dr2, dr3,
             g0, g1, s0, s1, zf32, xbf, bias_v, acc, xw_sp,
             gsem0, gsem1, ssem0, ssem1):
    cid = lax.axis_index("c")
    sid = lax.axis_index("s")
    pltpu.sync_copy(b_hbm.at[pl.ds(cid * FH, FH)], bias_v)

    pltpu.sync_copy(packed_hbm.at[sid], packed_v)
    pltpu.sync_copy(vals_hbm.at[sid], vals_v)

    zero = jnp.zeros((16,), jnp.float32)

    def zero_row(i, carry):
        for j in range(FH // 16):
            zf32[i, pl.ds(j * 16, 16)] = zero
        return carry

    lax.fori_loop(0, XCONV, zero_row, 0)
    for k in range(XROWS // XCONV):
        pltpu.sync_copy(zf32, acc.at[pl.ds(sid * XROWS + k * XCONV, XCONV)])

    for k in range(XROWS // XCONV):
        r0 = sid * XROWS + k * XCONV
        pltpu.sync_copy(xw_hbm.at[pl.ds(r0, XCONV), pl.ds(cid * FH, FH)],
                        zf32)

        def conv_row(r, carry):
            for j in range(FH // 32):
                a = zf32[r, pl.ds(j * 32, 16)]
                bb = zf32[r, pl.ds(j * 32 + 16, 16)]
                xbf[r, pl.ds(j * 32, 32)] = plsc.pack(
                    a, bb, format=plsc.PackFormat.INTERLEAVED)
            return carry

        lax.fori_loop(0, XCONV, conv_row, 0)
        pltpu.sync_copy(xbf, xw_sp.at[pl.ds(r0, XCONV)])
    plsc.subcore_barrier()

    srings = (sr0, sr1, sr2, sr3)
    drings = (dr0, dr1, dr2, dr3)
    gbufs = (g0, g1)
    sbufs = (s0, s1)
    gsems = (gsem0, gsem1)
    ssems = (ssem0, ssem1)

    def unpack_idx(c, r):
        for g in range(C // 16):
            sl = pl.ds(g * 16, 16)
            p = packed_v[c, sl]
            srings[r][sl] = p & 0x3FFF
            drings[r][sl] = p >> 14

    def start_gather(r, b):
        pltpu.async_copy(xw_sp.at[srings[r]], gbufs[b], gsems[b])

    def wait_gather(r, b):
        pltpu.make_async_copy(xw_sp.at[srings[r]], gbufs[b],
                              gsems[b]).wait()

    def start_scatter(r, b):
        pltpu.async_copy(sbufs[b], acc.at[drings[r]], ssems[b], add=True)

    def wait_scatter(r, b):
        pltpu.make_async_copy(sbufs[b], acc.at[drings[r]], ssems[b]).wait()

    def scale(c, b):
        gbuf, sbuf = gbufs[b], sbufs[b]
        for g in range(C // 16):
            vv = vals_v[c, pl.ds(g * 16, 16)]
            for l in range(16):
                v = vv[l]
                base = g * 16 + l
                for j in range(FH // 32):
                    x = gbuf[base, pl.ds(j * 32, 32)]
                    a, bb = plsc.unpack(
                        x, format=plsc.PackFormat.INTERLEAVED)
                    sbuf[base, pl.ds(j * 32, 16)] = a * v
                    sbuf[base, pl.ds(j * 32 + 16, 16)] = bb * v

    CH = CHUNKS_PER_TILE

    unpack_idx(0, 0)
    start_gather(0, 0)
    unpack_idx(1, 1)
    start_gather(1, 1)
    wait_gather(0, 0)
    scale(0, 0)
    start_scatter(0, 0)
    unpack_idx(2, 2)
    start_gather(2, 0)
    wait_gather(1, 1)
    scale(1, 1)
    start_scatter(1, 1)

    @pl.loop(2, CH - 4, step=4)
    def _quads(c):
        for k in range(4):
            cc = c + k
            rk = (2 + k) % 4
            rn = (3 + k) % 4
            bk = k % 2
            bn = (k + 1) % 2
            unpack_idx(cc + 1, rn)
            start_gather(rn, bn)
            wait_gather(rk, bk)
            wait_scatter(rk, bk)
            scale(cc, bk)
            start_scatter(rk, bk)

    for cc in range(CH - 4, CH):
        rk = cc % 4
        rn = (cc + 1) % 4
        bk = cc % 2
        bn = (cc + 1) % 2
        if cc + 1 < CH:
            unpack_idx(cc + 1, rn)
            start_gather(rn, bn)
        wait_gather(rk, bk)
        wait_scatter(rk, bk)
        scale(cc, bk)
        start_scatter(rk, bk)
    wait_scatter((CH - 2) % 4, (CH - 2) % 2)
    wait_scatter((CH - 1) % 4, (CH - 1) % 2)
    plsc.subcore_barrier()

    bvec = [bias_v[pl.ds(j * 16, 16)] for j in range(FH // 16)]
    for k in range(XROWS // XCONV):
        r0 = sid * XROWS + k * XCONV
        pltpu.sync_copy(acc.at[pl.ds(r0, XCONV)], zf32)

        def pub_row(r, carry):
            for j in range(FH // 16):
                sl = pl.ds(j * 16, 16)
                zf32[r, sl] = zf32[r, sl] + bvec[j]
            return carry

        lax.fori_loop(0, XCONV, pub_row, 0)
        pltpu.sync_copy(zf32, out_hbm.at[pl.ds(r0, XCONV),
                                         pl.ds(cid * FH, FH)])


_sc_scatter = functools.partial(
    pl.kernel,
    out_type=jax.ShapeDtypeStruct((N, F), jnp.float32),
    mesh=plsc.VectorSubcoreMesh(core_axis_name="c", subcore_axis_name="s"),
    compiler_params=pltpu.CompilerParams(use_tc_tiling_on_sc=False,
                                         needs_layout_passes=False),
    scratch_types=[
        pltpu.VMEM((CHUNKS_PER_TILE, C), jnp.int32),
        pltpu.VMEM((CHUNKS_PER_TILE, C), jnp.float32),
        pltpu.VMEM((C,), jnp.int32),
        pltpu.VMEM((C,), jnp.int32),
        pltpu.VMEM((C,), jnp.int32),
        pltpu.VMEM((C,), jnp.int32),
        pltpu.VMEM((C,), jnp.int32),
        pltpu.VMEM((C,), jnp.int32),
        pltpu.VMEM((C,), jnp.int32),
        pltpu.VMEM((C,), jnp.int32),
        pltpu.VMEM((C, FH), jnp.bfloat16),
        pltpu.VMEM((C, FH), jnp.bfloat16),
        pltpu.VMEM((C, FH), jnp.float32),
        pltpu.VMEM((C, FH), jnp.float32),
        pltpu.VMEM((XCONV, FH), jnp.float32),
        pltpu.VMEM((XCONV, FH), jnp.bfloat16),
        pltpu.VMEM((FH,), jnp.float32),
        pltpu.VMEM_SHARED((N, FH), jnp.float32),
        pltpu.VMEM_SHARED((N, FH), jnp.bfloat16),
        pltpu.SemaphoreType.DMA,
        pltpu.SemaphoreType.DMA,
        pltpu.SemaphoreType.DMA,
        pltpu.SemaphoreType.DMA,
    ],
)(_sc_body)


def kernel(X, edge_index, A_values, W, b):
    XW = _matmul(X, W)
    shape3 = (NS, CHUNKS_PER_TILE, C)
    ei = edge_index.astype(jnp.int32)
    packed = ((ei[0] << 14) | ei[1]).reshape(shape3)
    return _sc_scatter(XW, packed, A_values.reshape(shape3), b)
